# Initial kernel scaffold; baseline (speedup 1.0000x reference)
import jax
import jax.numpy as jnp
from jax.experimental import pallas as pl


def kernel(QR, syms, W, b):
    raise NotImplementedError("write your pallas kernel here")


# speedup vs baseline: 1.4696x; 1.4696x over previous
import functools

import jax
import jax.numpy as jnp
from jax import lax
from jax.experimental import pallas as pl
from jax.experimental.pallas import tpu as pltpu
from jax.experimental.pallas import tpu_sc as plsc

VOCAB = 73
VOCAB_PAD = 80
GLYPH_DIM = 256
OUT_DIM = 512
B, L = 1024, 200
NTOK = B * L
NWORKERS = 32
PER_W = NTOK // NWORKERS
CHUNK = 64
NCHUNK = PER_W // CHUNK


def _table_body(s_ref, w_ref, b_ref, o_ref):
    o_ref[...] = lax.dot_general(
        s_ref[...], w_ref[...],
        (((1,), (1,)), ((), ())),
        preferred_element_type=jnp.float32,
    ) + b_ref[...]


def _gather_body(table_hbm, idx_hbm, out_hbm, idx_v, rows_v, gsem):
    wid = lax.axis_index("s") * 2 + lax.axis_index("c")
    row_base = wid * PER_W
    pltpu.sync_copy(idx_hbm.at[wid], idx_v)

    def step(c, _):
        pltpu.async_copy(table_hbm.at[idx_v.at[c]], rows_v, gsem).wait()
        pltpu.sync_copy(rows_v, out_hbm.at[pl.ds(row_base + c * CHUNK, CHUNK)])
        return 0

    lax.fori_loop(0, NCHUNK, step, 0)


def kernel(QR, syms, W, b):
    syms_pad = jnp.pad(syms, ((0, VOCAB_PAD - VOCAB), (0, 0)))
    table = pl.pallas_call(
        _table_body,
        out_shape=jax.ShapeDtypeStruct((VOCAB_PAD, OUT_DIM), jnp.float32),
    )(syms_pad, W, b.reshape(1, OUT_DIM))

    idx = QR.reshape(NWORKERS, NCHUNK, CHUNK)
    mesh = plsc.VectorSubcoreMesh(core_axis_name="c", subcore_axis_name="s")
    gather = functools.partial(
        pl.kernel,
        out_type=jax.ShapeDtypeStruct((NTOK, OUT_DIM), jnp.float32),
        mesh=mesh,
        scratch_types=[
            pltpu.VMEM((NCHUNK, CHUNK), jnp.int32),
            pltpu.VMEM((CHUNK, OUT_DIM), jnp.float32),
            pltpu.SemaphoreType.DMA,
        ],
    )(_gather_body)
    out = gather(table, idx)
    return out.reshape(B, L, OUT_DIM)
